# baseline (speedup 1.0000x reference)
import functools

import jax
import jax.numpy as jnp
from jax import lax
from jax.experimental import pallas as pl
from jax.experimental.pallas import tpu as pltpu
from jax.experimental.pallas import tpu_sc as plsc

_M = 100000
_N = 16384
_D = 128

_NW = 32
_SLOTS = 3136
_MPAD = _NW * _SLOTS
_L = 16
_NVEC = _N // _L
_SVEC = _SLOTS // _L
_CHUNK = 112
_NCHUNK = _SLOTS // _CHUNK
_TSPAD = _N + 8


def _sc_body(nid_hbm, msgs_hbm, ts_hbm,
             mask_out, ts_out, msgs_out,
             nid_v, lane_buf, idx_v, mask_v, ts_all_v, ts_o_v,
             row_a, row_b, sem_a, sem_b):
    wid = lax.axis_index("s") * 2 + lax.axis_index("c")
    base = wid * _SLOTS

    pltpu.sync_copy(nid_hbm, nid_v)
    pltpu.sync_copy(ts_hbm, ts_all_v)

    lanes = lax.iota(jnp.int32, _L)
    lane_off = lanes * _SLOTS
    minus1 = jnp.full((_L,), -1, jnp.int32)

    def init_body(i, c):
        lane_buf[pl.ds(i * _L, _L)] = minus1
        return c
    lax.fori_loop(0, (_L * _SLOTS) // _L, init_body, 0)

    def scan_body(j, c):
        nids = nid_v[pl.ds(j * _L, _L)]
        pos = lanes + j * _L
        loc = nids - base
        inr = (loc >= 0) & (loc < _SLOTS)
        safe = jnp.where(inr, loc, 0)
        plsc.store_scatter(lane_buf, [lane_off + safe], pos, mask=inr)
        return c
    lax.fori_loop(0, _NVEC, scan_body, 0)

    def comb_body(v, c):
        o = v * _L
        acc = lane_buf[pl.ds(o, _L)]
        for l in range(1, _L):
            acc = jnp.maximum(acc, lane_buf[pl.ds(l * _SLOTS + o, _L)])
        upd = acc >= 0
        idx_v[pl.ds(o, _L)] = jnp.where(upd, acc, _N)
        mask_v[pl.ds(o, _L)] = jnp.where(upd, 1, 0).astype(jnp.int32)
        ts_o_v[pl.ds(o, _L)] = plsc.load_gather(
            ts_all_v, [jnp.where(upd, acc, _N)])
        return c
    lax.fori_loop(0, _SVEC, comb_body, 0)

    pltpu.sync_copy(mask_v, mask_out.at[pl.ds(base, _SLOTS)])
    pltpu.sync_copy(ts_o_v, ts_out.at[pl.ds(base, _SLOTS)])

    bufs = (row_a, row_b)
    sems = (sem_a, sem_b)
    cps = [None, None]
    cps[0] = pltpu.async_copy(
        msgs_hbm.at[idx_v.at[pl.ds(0, _CHUNK)]], row_a, sem_a)
    for c in range(_NCHUNK):
        cur = c % 2
        nxt = (c + 1) % 2
        if c + 1 < _NCHUNK:
            cps[nxt] = pltpu.async_copy(
                msgs_hbm.at[idx_v.at[pl.ds((c + 1) * _CHUNK, _CHUNK)]],
                bufs[nxt], sems[nxt])
        cps[cur].wait()
        pltpu.sync_copy(
            bufs[cur], msgs_out.at[pl.ds(base + c * _CHUNK, _CHUNK)])


@jax.jit
def _run(nid, msgs_p, ts_p):
    mesh = plsc.VectorSubcoreMesh(core_axis_name="c", subcore_axis_name="s")
    f = pl.kernel(
        _sc_body,
        out_type=[
            jax.ShapeDtypeStruct((_MPAD,), jnp.int32),
            jax.ShapeDtypeStruct((_MPAD,), jnp.float32),
            jax.ShapeDtypeStruct((_MPAD, _D), jnp.float32),
        ],
        mesh=mesh,
        compiler_params=pltpu.CompilerParams(needs_layout_passes=False),
        scratch_types=[
            pltpu.VMEM((_N,), jnp.int32),
            pltpu.VMEM((_L * _SLOTS,), jnp.int32),
            pltpu.VMEM((_SLOTS,), jnp.int32),
            pltpu.VMEM((_SLOTS,), jnp.int32),
            pltpu.VMEM((_TSPAD,), jnp.float32),
            pltpu.VMEM((_SLOTS,), jnp.float32),
            pltpu.VMEM((_CHUNK, _D), jnp.float32),
            pltpu.VMEM((_CHUNK, _D), jnp.float32),
            pltpu.SemaphoreType.DMA,
            pltpu.SemaphoreType.DMA,
        ],
    )
    return f(nid, msgs_p, ts_p)


def kernel(node_ids, msgs, ts):
    nid = node_ids.astype(jnp.int32)
    msgs_p = jnp.concatenate(
        [msgs, jnp.zeros((_TSPAD - _N, _D), msgs.dtype)], axis=0)
    ts_p = jnp.concatenate([ts, jnp.zeros((_TSPAD - _N,), ts.dtype)], axis=0)
    mask_i, ts_o, msgs_o = _run(nid, msgs_p, ts_p)
    return (mask_i[:_M].astype(bool), msgs_o[:_M], ts_o[:_M])
